# baseline (speedup 1.0000x reference)
import functools

import jax
import jax.numpy as jnp
from jax import lax
from jax.experimental import pallas as pl
from jax.experimental.pallas import tpu as pltpu
from jax.experimental.pallas import tpu_sc as plsc

_B, _R, _A, _S, _F = 16, 2, 4, 14, 4096
_N = _B * _R * _A
_BK = 8
_L = 16


def _w_scatter_body(n_sc, n_pad, n_rows, n_cores, n_workers,
                    c_hbm, idx_hbm, w_hbm, idx_v, val_v, row_v):
    wid = lax.axis_index("s") * n_cores + lax.axis_index("c")
    n_iter = (n_rows + n_workers - 1) // n_workers

    def task(t):
        zf = jnp.zeros((_L,), jnp.float32)

        def zero_body(j, carry):
            row_v[pl.ds(pl.multiple_of(j * _L, _L), _L)] = zf
            return carry

        lax.fori_loop(0, _F // _L, zero_body, 0)

        pltpu.sync_copy(idx_hbm.at[pl.ds(0, n_pad)], idx_v)
        pltpu.sync_copy(c_hbm.at[pl.ds(t * n_pad, n_pad)], val_v)

        def scat_body(k, carry):
            off = pl.multiple_of(k * _L, _L)
            idx = idx_v[pl.ds(off, _L)]
            val = val_v[pl.ds(off, _L)]
            mask = (lax.iota(jnp.int32, _L) + off) < n_sc
            plsc.store_scatter(row_v, [idx], val, mask=mask)
            return carry

        lax.fori_loop(0, (n_sc + _L - 1) // _L, scat_body, 0)
        pltpu.sync_copy(row_v, w_hbm.at[pl.ds(t * _F, _F)])

    for i in range(n_iter):
        t = wid + i * n_workers

        @pl.when(t < n_rows)
        def _():
            task(t)


def _build_weight_map(c2, sc_ind):
    n_rows, n_sc = c2.shape
    n_pad = (n_sc + 127) // 128 * 128
    c_flat = jnp.pad(c2, ((0, 0), (0, n_pad - n_sc))).reshape(-1)
    idx_flat = jnp.pad(sc_ind, (0, n_pad - n_sc))
    mesh = plsc.VectorSubcoreMesh(core_axis_name="c", subcore_axis_name="s")
    n_workers = mesh.num_cores * mesh.num_subcores
    body = functools.partial(_w_scatter_body, n_sc, n_pad, n_rows,
                             mesh.num_cores, n_workers)
    return pl.kernel(
        body,
        out_type=jax.ShapeDtypeStruct((n_rows * _F,), jnp.float32),
        mesh=mesh,
        scratch_types=[
            pltpu.VMEM((n_pad,), jnp.int32),
            pltpu.VMEM((n_pad,), jnp.float32),
            pltpu.VMEM((_F,), jnp.float32),
        ],
        compiler_params=pltpu.CompilerParams(needs_layout_passes=False),
    )(c_flat, idx_flat)


def _demash_body(ir_ref, ii_ref, w_ref, out_ref):
    ir = ir_ref[...]
    ii = ii_ref[...]
    wr = w_ref[0][None]
    wi = w_ref[1][None]
    out_ref[0] = ir * wr + ii * wi
    out_ref[1] = ii * wr - ir * wi


def kernel(inputs_real, inputs_imag, sc_ind, c_diag_real, c_diag_imag):
    n_sc = sc_ind.shape[0]
    ir = inputs_real.reshape(_N, _S, _F)
    ii = inputs_imag.reshape(_N, _S, _F)
    c2 = jnp.concatenate(
        [c_diag_real.reshape(_S, n_sc), c_diag_imag.reshape(_S, n_sc)], axis=0)
    w = _build_weight_map(c2, sc_ind).reshape(2, _S, _F)

    out = pl.pallas_call(
        _demash_body,
        grid=(_N // _BK,),
        in_specs=[
            pl.BlockSpec((_BK, _S, _F), lambda i: (i, 0, 0)),
            pl.BlockSpec((_BK, _S, _F), lambda i: (i, 0, 0)),
            pl.BlockSpec((2, _S, _F), lambda i: (0, 0, 0)),
        ],
        out_specs=pl.BlockSpec((2, _BK, _S, _F), lambda i: (0, i, 0, 0)),
        out_shape=jax.ShapeDtypeStruct((2, _N, _S, _F), jnp.float32),
    )(ir, ii, w)
    return out.reshape(2, _B, _R, _A, _S, _F)
